# Initial kernel scaffold; baseline (speedup 1.0000x reference)
import jax
import jax.numpy as jnp
from jax.experimental import pallas as pl


def kernel(seq, edge_index, edge_weight, W, bias, prelu_a):
    raise NotImplementedError("write your pallas kernel here")


# speedup vs baseline: 9.9626x; 9.9626x over previous
import functools

import jax
import jax.numpy as jnp
from jax import lax
from jax.experimental import pallas as pl
from jax.experimental.pallas import tpu as pltpu
from jax.experimental.pallas import tpu_sc as plsc

N = 10000
E = 320000
D = 128
OUT = 128

NC = 2
NS = 16
LANES = 16
DH = D // NC
K = 32
NCHUNK_TOT = E // K
NCHUNK = NCHUNK_TOT // NS
SB = 125
ROWS_PT = N // NS
ZROWS = 25


def _sc_diffusion_body(seq_h, src_h, dst_h, w_h, out_h,
                       xa, xb, srcb, dstb, wb, rows, zbuf, sem):
    c = lax.axis_index("c")
    s = lax.axis_index("s")
    col0 = c * DH
    row0 = s * ROWS_PT

    pltpu.sync_copy(seq_h.at[pl.ds(row0, ROWS_PT), pl.ds(col0, DH)],
                    xa.at[pl.ds(row0, ROWS_PT)])

    zv = jnp.zeros((LANES,), jnp.float32)
    for i in range(ZROWS):
        for v in range(DH // LANES):
            zbuf[i, pl.ds(v * LANES, LANES)] = zv

    def zero_acc(acc):
        def zbody(z, _):
            pltpu.sync_copy(zbuf, acc.at[pl.ds(row0 + z * ZROWS, ZROWS)])
            return _
        lax.fori_loop(0, ROWS_PT // ZROWS, zbody, None)

    def edge_pass(cur, acc):
        for blk in range(NCHUNK // SB):
            base = s * NCHUNK + blk * SB
            pltpu.sync_copy(src_h.at[pl.ds(base, SB)], srcb)
            pltpu.sync_copy(dst_h.at[pl.ds(base, SB)], dstb)
            pltpu.sync_copy(w_h.at[pl.ds(base, SB)], wb)

            def ebody(j, _):
                pltpu.async_copy(cur.at[srcb.at[j]], rows, sem).wait()
                wvs = [wb[j, pl.ds(g * LANES, LANES)]
                       for g in range(K // LANES)]
                for r in range(K):
                    wsc = wvs[r // LANES][r % LANES]
                    for v in range(DH // LANES):
                        sl = pl.ds(v * LANES, LANES)
                        rows[r, sl] = rows[r, sl] * wsc
                pltpu.sync_copy(rows, acc.at[dstb.at[j]], add=True)
                return _
            lax.fori_loop(0, SB, ebody, None)

    bufs = (xa, xb)
    for step in range(4):
        cur = bufs[step % 2]
        acc = bufs[1 - step % 2]
        zero_acc(acc)
        plsc.subcore_barrier()
        edge_pass(cur, acc)
        plsc.subcore_barrier()
        pltpu.sync_copy(acc.at[pl.ds(row0, ROWS_PT)],
                        out_h.at[step, pl.ds(row0, ROWS_PT), pl.ds(col0, DH)])


def _sc_diffusion(seq, src2d, dst2d, w2d):
    mesh = plsc.VectorSubcoreMesh(core_axis_name="c", subcore_axis_name="s",
                                  num_cores=NC, num_subcores=NS)
    fn = pl.kernel(
        _sc_diffusion_body,
        out_type=jax.ShapeDtypeStruct((4, N, D), jnp.float32),
        mesh=mesh,
        scratch_types=[
            pltpu.VMEM_SHARED((N, DH), jnp.float32),
            pltpu.VMEM_SHARED((N, DH), jnp.float32),
            pltpu.VMEM((SB, K), jnp.int32),
            pltpu.VMEM((SB, K), jnp.int32),
            pltpu.VMEM((SB, K), jnp.float32),
            pltpu.VMEM((K, DH), jnp.float32),
            pltpu.VMEM((ZROWS, DH), jnp.float32),
            pltpu.SemaphoreType.DMA,
        ],
        compiler_params=pltpu.CompilerParams(use_tc_tiling_on_sc=False),
    )
    return fn(seq, src2d, dst2d, w2d)


BLK = 1000


def _tc_head_body(x_ref, w_ref, b_ref, a_ref, o_ref):
    x1 = x_ref[0]
    x2 = x_ref[1]
    x3 = x_ref[2]
    x4 = x_ref[3]
    d12 = jnp.abs(x1 - x2)
    d24 = jnp.abs(x2 - x4)
    d32 = jnp.abs(x3 - x2)
    d43 = jnp.abs(x4 - x3)
    acc = jnp.dot(x4, w_ref[0], preferred_element_type=jnp.float32)
    acc = acc + jnp.dot(d12, w_ref[1], preferred_element_type=jnp.float32)
    acc = acc + jnp.dot(d24, w_ref[2], preferred_element_type=jnp.float32)
    acc = acc + jnp.dot(d32, w_ref[3], preferred_element_type=jnp.float32)
    acc = acc + jnp.dot(d43, w_ref[4], preferred_element_type=jnp.float32)
    y = acc + b_ref[0][None, :]
    a = a_ref[0, 0]
    o_ref[...] = jnp.where(y >= 0, y, a * y)


def _tc_head(xs, wstack, bias, prelu_a):
    return pl.pallas_call(
        _tc_head_body,
        grid=(N // BLK,),
        in_specs=[
            pl.BlockSpec((4, BLK, D), lambda i: (0, i, 0)),
            pl.BlockSpec((5, D, OUT), lambda i: (0, 0, 0)),
            pl.BlockSpec((1, OUT), lambda i: (0, 0)),
            pl.BlockSpec((1, 1), lambda i: (0, 0)),
        ],
        out_specs=pl.BlockSpec((BLK, OUT), lambda i: (i, 0)),
        out_shape=jax.ShapeDtypeStruct((N, OUT), jnp.float32),
    )(xs, wstack, bias, prelu_a)


def kernel(seq, edge_index, edge_weight, W, bias, prelu_a):
    src2d = edge_index[0].astype(jnp.int32).reshape(NCHUNK_TOT, K)
    dst2d = edge_index[1].astype(jnp.int32).reshape(NCHUNK_TOT, K)
    w2d = edge_weight.reshape(NCHUNK_TOT, K)

    xs = _sc_diffusion(seq, src2d, dst2d, w2d)

    w0 = W[:, 0 * D:1 * D].T
    w13 = (W[:, 1 * D:2 * D] + W[:, 3 * D:4 * D]).T
    w2 = W[:, 2 * D:3 * D].T
    w4 = W[:, 4 * D:5 * D].T
    w5 = W[:, 5 * D:6 * D].T
    wstack = jnp.stack([w0, w13, w2, w4, w5])

    return _tc_head(xs, wstack, bias.reshape(1, OUT),
                    prelu_a.reshape(1, 1).astype(jnp.float32))
